# baseline (speedup 1.0000x reference)
import jax
import jax.numpy as jnp
import numpy as np
from jax.experimental import pallas as pl

_B, _N, _D = 4, 256, 64
_H, _HD = 4, 64


def _gat_body(x_ref, pool_ref, W0, as0, ad0, b0, W1, as1, ad1, b1,
              W2, as2, ad2, b2, Wr, br, out_ref):
    B, N, H, C = _B, _N, _H, _HD
    x = x_ref[...]
    for (W, a_s, a_d, b) in ((W0, as0, ad0, b0),
                             (W1, as1, ad1, b1),
                             (W2, as2, ad2, b2)):
        xp = jnp.dot(x, W[...], preferred_element_type=jnp.float32)
        td = xp * a_d[...]
        outs = []
        for bi in range(B):
            r0 = bi * N
            acc = jnp.zeros((N, C), dtype=jnp.float32)
            for h in range(H):
                xpbh = xp[r0:r0 + N, h * C:(h + 1) * C]
                dh = td[r0:r0 + N, h * C:(h + 1) * C].sum(
                    axis=-1, keepdims=True)
                sh = jax.lax.dot_general(
                    a_s[:, h * C:(h + 1) * C], xpbh,
                    dimension_numbers=(((1,), (1,)), ((), ())),
                    preferred_element_type=jnp.float32)
                z = dh + sh
                logits = jnp.maximum(z, 0.2 * z)
                m = jnp.max(logits, axis=-1, keepdims=True)
                e = jnp.exp(logits - m)
                den = jnp.sum(e, axis=-1, keepdims=True)
                num = jnp.dot(e, xpbh, preferred_element_type=jnp.float32)
                acc = acc + num * (1.0 / (den + 1e-16))
            outs.append(acc)
        x = jax.nn.relu(jnp.concatenate(outs, axis=0) * (1.0 / H) + b[...])
    pooled = jnp.dot(pool_ref[...], x, preferred_element_type=jnp.float32)
    out_ref[...] = (jnp.dot(pooled, Wr[...], preferred_element_type=jnp.float32)
                    + br[...])


def kernel(embeddings, W0, as0, ad0, b0, W1, as1, ad1, b1, W2, as2, ad2, b2,
           Wr, br):
    as0, ad0 = as0.reshape(1, -1), ad0.reshape(1, -1)
    as1, ad1 = as1.reshape(1, -1), ad1.reshape(1, -1)
    as2, ad2 = as2.reshape(1, -1), ad2.reshape(1, -1)
    xflat = embeddings.reshape(_B * _N, _D)
    pool = jnp.asarray(
        np.kron(np.eye(_B, dtype=np.float32),
                np.full((1, _N), 1.0 / _N, dtype=np.float32)))
    return pl.pallas_call(
        _gat_body,
        out_shape=jax.ShapeDtypeStruct((_B, _D), jnp.float32),
    )(xflat, pool, W0, as0, ad0, b0, W1, as1, ad1, b1, W2, as2, ad2, b2,
      Wr, br)
